# baseline (speedup 1.0000x reference)
import functools

import jax
import jax.numpy as jnp
from jax import lax
from jax.experimental import pallas as pl
from jax.experimental import pallas as pl
from jax.experimental.pallas import tpu as pltpu
from jax.experimental.pallas import tpu_sc as plsc

_B, _C, _T = 16, 256, 1024
_K, _E, _G = 1024, 128, 2

_NW = 32
_EQ = 32
_NQ = _E // _EQ
_PPW = _B * _NQ // _NW


def _argmin_body(x_ref, cb_ref, idx_ref):
    xb = x_ref[0]
    cb = cb_ref[...]
    e2 = jnp.sum(cb * cb, axis=1)
    x2 = jnp.sum(xb * xb, axis=0)
    xe = lax.dot_general(cb, xb, (((1,), (0,)), ((), ())),
                         preferred_element_type=jnp.float32)
    s = (x2[None, :] + e2[:, None]) - 2.0 * xe
    m = jnp.min(s, axis=0)
    kio = lax.broadcasted_iota(jnp.int32, (_K, _T), 0)
    idx_ref[0, 0, 0] = jnp.min(
        jnp.where(s == m[None, :], kio, jnp.int32(_K)), axis=0)


def _tc_argmin_group(xin, codebook, g):
    return pl.pallas_call(
        _argmin_body,
        grid=(_B,),
        in_specs=[
            pl.BlockSpec((1, _E, _T), lambda b: (b, 0, g)),
            pl.BlockSpec((_K, _E), lambda b: (0, 0)),
        ],
        out_specs=pl.BlockSpec((1, 1, 1, _T), lambda b: (b, 0, 0, 0)),
        out_shape=jax.ShapeDtypeStruct((_B, 1, 1, _T), jnp.int32),
        compiler_params=pltpu.CompilerParams(
            dimension_semantics=("parallel",),
        ),
    )(xin, codebook)


def _sc_gather_body(cbq_hbm, idx_hbm, out_hbm, cb_v, idx_v, out_v, sem):
    wid = lax.axis_index("s") * 2 + lax.axis_index("c")
    q = wid % _NQ
    pg = wid // _NQ
    pltpu.sync_copy(cbq_hbm.at[q], cb_v)
    copies = [None, None]
    for i in range(_PPW):
        p = pg + 8 * i
        buf = i % 2
        pltpu.sync_copy(idx_hbm.at[p], idx_v)

        if copies[buf] is not None:
            copies[buf].wait()

        @plsc.parallel_loop(0, (_T // 16) * _EQ, unroll=16)
        def _gather(i2):
            j = lax.shift_right_logical(i2, 5)
            e = lax.bitwise_and(i2, _EQ - 1)
            fidx = idx_v[pl.ds(j * 16, 16)] + lax.shift_left(e, 10)
            out_v[buf, e, pl.ds(j * 16, 16)] = plsc.load_gather(cb_v, [fidx])

        copies[buf] = pltpu.async_copy(
            out_v.at[buf], out_hbm.at[p, pl.ds(q * _EQ, _EQ), :], sem)
    for c in copies:
        if c is not None:
            c.wait()


def _sc_gather(cbq, idx2d):
    mesh = plsc.VectorSubcoreMesh(core_axis_name="c", subcore_axis_name="s")
    k = functools.partial(
        pl.kernel,
        mesh=mesh,
        out_type=jax.ShapeDtypeStruct((_B, _E, _T), jnp.float32),
        scratch_types=[
            pltpu.VMEM((_K * _EQ,), jnp.float32),
            pltpu.VMEM((_T,), jnp.int32),
            pltpu.VMEM((2, _EQ, _T), jnp.float32),
            pltpu.SemaphoreType.DMA,
        ],
        compiler_params=pltpu.CompilerParams(needs_layout_passes=False),
    )(_sc_gather_body)
    return k(cbq, idx2d)


def kernel(x, codebook):
    xin = x.reshape(_B, _E, _G * _T)
    cbq = codebook.T.reshape(_NQ, _EQ * _K)
    idx0 = _tc_argmin_group(xin, codebook, 0)
    q0 = _sc_gather(cbq, idx0.reshape(_B, _T))
    idx1 = _tc_argmin_group(xin, codebook, 1)
    q1 = _sc_gather(cbq, idx1.reshape(_B, _T))
    quantized = jnp.concatenate([q0, q1], axis=1).reshape(_B, _C, _T)
    indexes = jnp.stack([idx0.reshape(_B, _T), idx1.reshape(_B, _T)], axis=0)
    return quantized, indexes


# speedup vs baseline: 1.0575x; 1.0575x over previous
import functools

import jax
import jax.numpy as jnp
from jax import lax
from jax.experimental import pallas as pl
from jax.experimental.pallas import tpu as pltpu
from jax.experimental.pallas import tpu_sc as plsc

_B, _C, _T = 16, 256, 1024
_K, _E, _G = 1024, 128, 2
_TT = 1024
_P = _T // _TT

_NW = 32
_EQ = 32
_NQ = _E // _EQ
_NPAIR = _B * _G
_PAIRS_PER_W = _NPAIR // (_NW // _NQ)


def _argmin_body(x_ref, cb2_ref, e2_ref, idx_ref):
    xb = x_ref[0]
    x2 = jnp.sum(xb * xb, axis=0)
    xe2 = lax.dot_general(cb2_ref[...], xb, (((1,), (0,)), ((), ())),
                          preferred_element_type=jnp.float32)
    s = (x2[None, :] + e2_ref[...]) - xe2
    m = jnp.min(s, axis=0)
    kio = lax.broadcasted_iota(jnp.int32, (_K, _TT), 0)
    idx_ref[0, 0, 0] = jnp.min(
        jnp.where(s == m[None, :], kio, jnp.int32(_K)), axis=0)


def _tc_argmin(xin, cb2, e2):
    grid = (_B, _G, _P)
    return pl.pallas_call(
        _argmin_body,
        grid=grid,
        in_specs=[
            pl.BlockSpec((1, _E, _TT), lambda b, g, p: (b, 0, g * _P + p)),
            pl.BlockSpec((_K, _E), lambda b, g, p: (0, 0)),
            pl.BlockSpec((_K, 1), lambda b, g, p: (0, 0)),
        ],
        out_specs=pl.BlockSpec((1, 1, 1, _TT), lambda b, g, p: (b, g, 0, p)),
        out_shape=jax.ShapeDtypeStruct((_B, _G, 1, _T), jnp.int32),
        compiler_params=pltpu.CompilerParams(
            dimension_semantics=("parallel", "parallel", "parallel"),
        ),
    )(xin, cb2, e2)


def _sc_gather_body(cbq_hbm, idx_hbm, out_hbm, cb_v, idx_v, out_v, sem):
    wid = lax.axis_index("s") * 2 + lax.axis_index("c")
    q = wid % _NQ
    pg = wid // _NQ
    pltpu.sync_copy(cbq_hbm.at[q], cb_v)
    copies = [None, None]
    for i in range(_PAIRS_PER_W):
        p = pg + 8 * i
        buf = i % 2
        pltpu.sync_copy(idx_hbm.at[p], idx_v)

        if copies[buf] is not None:
            copies[buf].wait()

        @plsc.parallel_loop(0, (_T // 16) * _EQ, unroll=16)
        def _gather(i2):
            j = lax.shift_right_logical(i2, 5)
            e = lax.bitwise_and(i2, _EQ - 1)
            fidx = idx_v[pl.ds(j * 16, 16)] + lax.shift_left(e, 10)
            out_v[buf, e, pl.ds(j * 16, 16)] = plsc.load_gather(cb_v, [fidx])

        copies[buf] = pltpu.async_copy(
            out_v.at[buf], out_hbm.at[p, pl.ds(q * _EQ, _EQ), :], sem)
    for c in copies:
        c.wait()


def _sc_gather(cbq, idx2d):
    mesh = plsc.VectorSubcoreMesh(core_axis_name="c", subcore_axis_name="s")
    k = functools.partial(
        pl.kernel,
        mesh=mesh,
        out_type=jax.ShapeDtypeStruct((_NPAIR, _E, _T), jnp.float32),
        scratch_types=[
            pltpu.VMEM((_K * _EQ,), jnp.float32),
            pltpu.VMEM((_T,), jnp.int32),
            pltpu.VMEM((2, _EQ, _T), jnp.float32),
            pltpu.SemaphoreType.DMA,
        ],
        compiler_params=pltpu.CompilerParams(needs_layout_passes=False),
    )(_sc_gather_body)
    return k(cbq, idx2d)


def kernel(x, codebook):
    xin = x.reshape(_B, _E, _G * _T)
    e2 = jnp.sum(codebook ** 2, axis=1).reshape(_K, 1)
    idx = _tc_argmin(xin, codebook + codebook, e2)
    cbq = codebook.T.reshape(_NQ, _EQ * _K)
    q = _sc_gather(cbq, idx.reshape(_NPAIR, _T))
    quantized = q.reshape(_B, _C, _T)
    indexes = idx.reshape(_B, _G, _T).transpose(1, 0, 2)
    return quantized, indexes


# speedup vs baseline: 1.0787x; 1.0200x over previous
import functools

import jax
import jax.numpy as jnp
from jax import lax
from jax.experimental import pallas as pl
from jax.experimental.pallas import tpu as pltpu
from jax.experimental.pallas import tpu_sc as plsc

_B, _C, _T = 16, 256, 1024
_K, _E, _G = 1024, 128, 2
_TT = 1024
_P = _T // _TT

_NW = 32
_EQ = 32
_NQ = _E // _EQ
_NPAIR = _B * _G
_PAIRS_PER_W = _NPAIR // (_NW // _NQ)


def _argmin_body(x_ref, cb_ref, idx_ref):
    xb = x_ref[0]
    cb = cb_ref[...]
    e2 = jnp.sum(cb * cb, axis=1)
    x2 = jnp.sum(xb * xb, axis=0)
    xe = lax.dot_general(cb, xb, (((1,), (0,)), ((), ())),
                         preferred_element_type=jnp.float32)
    s = (x2[None, :] + e2[:, None]) - 2.0 * xe
    m = jnp.min(s, axis=0)
    kio = lax.broadcasted_iota(jnp.int32, (_K, _TT), 0)
    idx_ref[0, 0, 0] = jnp.min(
        jnp.where(s == m[None, :], kio, jnp.int32(_K)), axis=0)


def _tc_argmin(xin, codebook):
    grid = (_B, _G, _P)
    return pl.pallas_call(
        _argmin_body,
        grid=grid,
        in_specs=[
            pl.BlockSpec((1, _E, _TT), lambda b, g, p: (b, 0, g * _P + p)),
            pl.BlockSpec((_K, _E), lambda b, g, p: (0, 0)),
        ],
        out_specs=pl.BlockSpec((1, 1, 1, _TT), lambda b, g, p: (b, g, 0, p)),
        out_shape=jax.ShapeDtypeStruct((_B, _G, 1, _T), jnp.int32),
        compiler_params=pltpu.CompilerParams(
            dimension_semantics=("parallel", "parallel", "parallel"),
        ),
    )(xin, codebook)


def _sc_gather_body(cbq_hbm, idx_hbm, out_hbm, cb_v, idx_v, out_v, sem):
    wid = lax.axis_index("s") * 2 + lax.axis_index("c")
    q = wid % _NQ
    pg = wid // _NQ
    pltpu.sync_copy(cbq_hbm.at[q], cb_v)
    copies = [None, None]
    for i in range(_PAIRS_PER_W):
        p = pg + 8 * i
        buf = i % 2
        pltpu.sync_copy(idx_hbm.at[p], idx_v)

        if copies[buf] is not None:
            copies[buf].wait()

        @plsc.parallel_loop(0, (_T // 16) * _EQ, unroll=16)
        def _gather(i2):
            j = lax.shift_right_logical(i2, 5)
            e = lax.bitwise_and(i2, _EQ - 1)
            fidx = idx_v[pl.ds(j * 16, 16)] + lax.shift_left(e, 10)
            out_v[buf, e, pl.ds(j * 16, 16)] = plsc.load_gather(cb_v, [fidx])

        copies[buf] = pltpu.async_copy(
            out_v.at[buf], out_hbm.at[p, pl.ds(q * _EQ, _EQ), :], sem)
    for c in copies:
        c.wait()


def _sc_gather(cbq, idx2d):
    mesh = plsc.VectorSubcoreMesh(core_axis_name="c", subcore_axis_name="s")
    k = functools.partial(
        pl.kernel,
        mesh=mesh,
        out_type=jax.ShapeDtypeStruct((_NPAIR, _E, _T), jnp.float32),
        scratch_types=[
            pltpu.VMEM((_K * _EQ,), jnp.float32),
            pltpu.VMEM((_T,), jnp.int32),
            pltpu.VMEM((2, _EQ, _T), jnp.float32),
            pltpu.SemaphoreType.DMA,
        ],
        compiler_params=pltpu.CompilerParams(needs_layout_passes=False),
    )(_sc_gather_body)
    return k(cbq, idx2d)


def kernel(x, codebook):
    xin = x.reshape(_B, _E, _G * _T)
    idx = _tc_argmin(xin, codebook)
    cbq = codebook.T.reshape(_NQ, _EQ * _K)
    q = _sc_gather(cbq, idx.reshape(_NPAIR, _T))
    quantized = q.reshape(_B, _C, _T)
    indexes = idx.reshape(_B, _G, _T).transpose(1, 0, 2)
    return quantized, indexes


# speedup vs baseline: 1.1099x; 1.0289x over previous
import functools

import jax
import jax.numpy as jnp
from jax import lax
from jax.experimental import pallas as pl
from jax.experimental.pallas import tpu as pltpu
from jax.experimental.pallas import tpu_sc as plsc

_B, _C, _T = 16, 256, 1024
_K, _E, _G = 1024, 128, 2
_TT = 2048

_NW = 32
_EQ = 32
_NQ = _E // _EQ
_NPAIR = _B * _G
_PAIRS_PER_W = _NPAIR // (_NW // _NQ)


def _argmin_body(x_ref, cb_ref, idx_ref):
    xb = x_ref[0]
    cb = cb_ref[...]
    e2 = jnp.sum(cb * cb, axis=1)
    x2 = jnp.sum(xb * xb, axis=0)
    xe = lax.dot_general(cb, xb, (((1,), (0,)), ((), ())),
                         preferred_element_type=jnp.float32)
    s = (x2[None, :] + e2[:, None]) - 2.0 * xe
    m = jnp.min(s, axis=0)
    kio = lax.broadcasted_iota(jnp.int32, (_K, _TT), 0)
    idx_ref[0, 0, 0] = jnp.min(
        jnp.where(s == m[None, :], kio, jnp.int32(_K)), axis=0)


def _tc_argmin(xin, codebook):
    return pl.pallas_call(
        _argmin_body,
        grid=(_B,),
        in_specs=[
            pl.BlockSpec((1, _E, _TT), lambda b: (b, 0, 0)),
            pl.BlockSpec((_K, _E), lambda b: (0, 0)),
        ],
        out_specs=pl.BlockSpec((1, 1, 1, _TT), lambda b: (b, 0, 0, 0)),
        out_shape=jax.ShapeDtypeStruct((_B, 1, 1, _TT), jnp.int32),
        compiler_params=pltpu.CompilerParams(
            dimension_semantics=("parallel",),
        ),
    )(xin, codebook)


def _sc_gather_body(cbq_hbm, idx_hbm, out_hbm, cb_v, idx_v, out_v, sem):
    wid = lax.axis_index("s") * 2 + lax.axis_index("c")
    q = wid % _NQ
    pg = wid // _NQ
    pltpu.sync_copy(cbq_hbm.at[q], cb_v)
    copies = [None, None]
    for i in range(_PAIRS_PER_W):
        p = pg + 8 * i
        buf = i % 2
        pltpu.sync_copy(idx_hbm.at[p], idx_v)

        if copies[buf] is not None:
            copies[buf].wait()

        @plsc.parallel_loop(0, (_T // 16) * _EQ, unroll=16)
        def _gather(i2):
            j = lax.shift_right_logical(i2, 5)
            e = lax.bitwise_and(i2, _EQ - 1)
            fidx = idx_v[pl.ds(j * 16, 16)] + lax.shift_left(e, 10)
            out_v[buf, e, pl.ds(j * 16, 16)] = plsc.load_gather(cb_v, [fidx])

        copies[buf] = pltpu.async_copy(
            out_v.at[buf], out_hbm.at[p, pl.ds(q * _EQ, _EQ), :], sem)
    for c in copies:
        c.wait()


def _sc_gather(cbq, idx2d):
    mesh = plsc.VectorSubcoreMesh(core_axis_name="c", subcore_axis_name="s")
    k = functools.partial(
        pl.kernel,
        mesh=mesh,
        out_type=jax.ShapeDtypeStruct((_NPAIR, _E, _T), jnp.float32),
        scratch_types=[
            pltpu.VMEM((_K * _EQ,), jnp.float32),
            pltpu.VMEM((_T,), jnp.int32),
            pltpu.VMEM((2, _EQ, _T), jnp.float32),
            pltpu.SemaphoreType.DMA,
        ],
        compiler_params=pltpu.CompilerParams(needs_layout_passes=False),
    )(_sc_gather_body)
    return k(cbq, idx2d)


def kernel(x, codebook):
    xin = x.reshape(_B, _E, _G * _T)
    idx = _tc_argmin(xin, codebook)
    cbq = codebook.T.reshape(_NQ, _EQ * _K)
    q = _sc_gather(cbq, idx.reshape(_NPAIR, _T))
    quantized = q.reshape(_B, _C, _T)
    indexes = idx.reshape(_B, _G, _T).transpose(1, 0, 2)
    return quantized, indexes
